# baseline (speedup 1.0000x reference)
import functools

import jax
import jax.numpy as jnp
from jax import lax
from jax.experimental import pallas as pl
from jax.experimental.pallas import tpu as pltpu, tpu_sc as plsc

N_NODES = 10000
D = 128
N_EDGES = 320000

NC = 2
NS = 16
NW = NC * NS

C = 128
NH = 2
NCHUNK = -(-(N_EDGES // NW) // (C * NH)) * NH
NCHUNK_H = NCHUNK // NH
EPT = NCHUNK * C
E_PAD = EPT * NW

RPT = -(-N_NODES // (NS * 8)) * 8
ACC_ROWS = RPT * NS


def _mm_body(x_ref, w_ref, o_ref):
    o_ref[...] = jnp.dot(x_ref[...], w_ref[...],
                         preferred_element_type=jnp.float32)


def _matmul(x, w):
    m_blk = 2000
    return pl.pallas_call(
        _mm_body,
        grid=(N_NODES // m_blk,),
        in_specs=[pl.BlockSpec((m_blk, D), lambda i: (i, 0)),
                  pl.BlockSpec((D, D), lambda i: (0, 0))],
        out_specs=pl.BlockSpec((m_blk, D), lambda i: (i, 0)),
        out_shape=jax.ShapeDtypeStruct((N_NODES, D), jnp.float32),
    )(x, w)


def _spmm_body(support_hbm, src_hbm, dst_hbm, ew_hbm, out_hbm,
               src_v, dst_v, ew_v, rows_a, rows_b, acc_sh,
               sem_a, sem_b, sem_sa, sem_sb):
    cid = lax.axis_index("c")
    sid = lax.axis_index("s")
    wid = cid * NS + sid
    rows_v = rows_a

    def _zrow(i, _):
        for j in range(D // 16):
            rows_v[i, pl.ds(j * 16, 16)] = jnp.zeros((16,), jnp.float32)
        return 0
    lax.fori_loop(0, C, _zrow, 0)

    zbase = sid * RPT
    off = 0
    while off < RPT:
        n = min(C, RPT - off)
        pltpu.sync_copy(rows_v.at[pl.ds(0, n)],
                        acc_sh.at[pl.ds(zbase + off, n)])
        off += n
    plsc.subcore_barrier()

    def _pass(h, _):
        hs = pl.ds(h * NCHUNK_H, NCHUNK_H)
        pltpu.sync_copy(src_hbm.at[wid].at[hs], src_v)
        pltpu.sync_copy(dst_hbm.at[wid].at[hs], dst_v)
        pltpu.sync_copy(ew_hbm.at[wid].at[hs], ew_v)

        def _iter(k2, _):
            c0 = k2 * 2

            ga = pltpu.async_copy(support_hbm.at[src_v.at[c0]],
                                  rows_a, sem_a)
            gb = pltpu.async_copy(support_hbm.at[src_v.at[c0 + 1]],
                                  rows_b, sem_b)

            def _scale(buf, c):
                def body(g, _):
                    ew16 = ew_v[c, pl.ds(g * 16, 16)]
                    for i in range(16):
                        s = ew16[i]
                        e = g * 16 + i
                        for j in range(D // 16):
                            sl = pl.ds(j * 16, 16)
                            buf[e, sl] = buf[e, sl] * s
                    return 0
                lax.fori_loop(0, C // 16, body, 0)

            ga.wait()
            _scale(rows_a, c0)
            gb.wait()
            _scale(rows_b, c0 + 1)

            sa = pltpu.async_copy(rows_a, acc_sh.at[dst_v.at[c0]],
                                  sem_sa, add=True)
            sb = pltpu.async_copy(rows_b, acc_sh.at[dst_v.at[c0 + 1]],
                                  sem_sb, add=True)
            sa.wait()
            sb.wait()
            return 0
        lax.fori_loop(0, NCHUNK_H // 2, _iter, 0)
        return 0
    lax.fori_loop(0, NH, _pass, 0)

    plsc.subcore_barrier()

    off = 0
    while off < RPT:
        n = min(C, RPT - off)
        pltpu.sync_copy(acc_sh.at[pl.ds(zbase + off, n)],
                        rows_v.at[pl.ds(0, n)])
        pltpu.sync_copy(rows_v.at[pl.ds(0, n)],
                        out_hbm.at[cid].at[pl.ds(zbase + off, n)])
        off += n


_spmm = functools.partial(
    pl.kernel,
    out_type=jax.ShapeDtypeStruct((NC, ACC_ROWS, D), jnp.float32),
    mesh=plsc.VectorSubcoreMesh(core_axis_name="c", subcore_axis_name="s"),
    scratch_types=[
        pltpu.VMEM((NCHUNK_H, C), jnp.int32),
        pltpu.VMEM((NCHUNK_H, C), jnp.int32),
        pltpu.VMEM((NCHUNK_H, C), jnp.float32),
        pltpu.VMEM((C, D), jnp.float32),
        pltpu.VMEM((C, D), jnp.float32),
        pltpu.VMEM_SHARED((ACC_ROWS, D), jnp.float32),
        pltpu.SemaphoreType.DMA,
        pltpu.SemaphoreType.DMA,
        pltpu.SemaphoreType.DMA,
        pltpu.SemaphoreType.DMA,
    ],
)(_spmm_body)


def _combine_body(p_ref, o_ref):
    o_ref[...] = jnp.maximum(p_ref[0] + p_ref[1], 0.0)


def _combine(partial):
    m_blk = 2000
    return pl.pallas_call(
        _combine_body,
        grid=(N_NODES // m_blk,),
        in_specs=[pl.BlockSpec((NC, m_blk, D), lambda i: (0, i, 0))],
        out_specs=pl.BlockSpec((m_blk, D), lambda i: (i, 0)),
        out_shape=jax.ShapeDtypeStruct((N_NODES, D), jnp.float32),
    )(partial)


def kernel(input, edge_index, edge_weight, weight):
    support = _matmul(input.astype(jnp.float32), weight.astype(jnp.float32))

    pad = E_PAD - N_EDGES
    src = jnp.pad(edge_index[0].astype(jnp.int32), (0, pad)).reshape(NW, NCHUNK, C)
    dst = jnp.pad(edge_index[1].astype(jnp.int32), (0, pad)).reshape(NW, NCHUNK, C)
    ew = jnp.pad(edge_weight.astype(jnp.float32), (0, pad)).reshape(NW, NCHUNK, C)

    partial = _spmm(support, src, dst, ew)
    return _combine(partial)


# speedup vs baseline: 1.3933x; 1.3933x over previous
import functools

import jax
import jax.numpy as jnp
from jax import lax
from jax.experimental import pallas as pl
from jax.experimental.pallas import tpu as pltpu, tpu_sc as plsc

N_NODES = 10000
D = 128
N_EDGES = 320000

NC = 2
NS = 16
NW = NC * NS

C = 128
NH = 1
NCHUNK = -(-(N_EDGES // NW) // (C * NH)) * NH
NCHUNK_H = NCHUNK // NH
EPT = NCHUNK * C
E_PAD = EPT * NW

RPT = -(-N_NODES // (NS * 8)) * 8
ACC_ROWS = RPT * NS


def _mm_body(x_ref, w_ref, o_ref):
    o_ref[...] = jnp.dot(x_ref[...], w_ref[...],
                         preferred_element_type=jnp.float32)


def _matmul(x, w):
    m_blk = 2000
    return pl.pallas_call(
        _mm_body,
        grid=(N_NODES // m_blk,),
        in_specs=[pl.BlockSpec((m_blk, D), lambda i: (i, 0)),
                  pl.BlockSpec((D, D), lambda i: (0, 0))],
        out_specs=pl.BlockSpec((m_blk, D), lambda i: (i, 0)),
        out_shape=jax.ShapeDtypeStruct((N_NODES, D), jnp.float32),
    )(x, w)


def _spmm_body(support_hbm, src_hbm, dst_hbm, ew_hbm, out_hbm,
               src_v, dst_v, ew_v, rows_v, acc_sh, sem):
    cid = lax.axis_index("c")
    sid = lax.axis_index("s")
    wid = cid * NS + sid

    def _zrow(i, _):
        for j in range(D // 16):
            rows_v[i, pl.ds(j * 16, 16)] = jnp.zeros((16,), jnp.float32)
        return 0
    lax.fori_loop(0, C, _zrow, 0)

    zbase = sid * RPT
    off = 0
    while off < RPT:
        n = min(C, RPT - off)
        pltpu.sync_copy(rows_v.at[pl.ds(0, n)],
                        acc_sh.at[pl.ds(zbase + off, n)])
        off += n
    plsc.subcore_barrier()

    pltpu.sync_copy(src_hbm.at[wid], src_v)
    pltpu.sync_copy(dst_hbm.at[wid], dst_v)
    pltpu.sync_copy(ew_hbm.at[wid], ew_v)

    def _chunk(k, _):
        pltpu.async_copy(support_hbm.at[src_v.at[k]], rows_v, sem).wait()

        def _scale(g, _):
            ew16 = ew_v[k, pl.ds(g * 16, 16)]
            for i in range(16):
                s = ew16[i]
                e = g * 16 + i
                for j in range(D // 16):
                    sl = pl.ds(j * 16, 16)
                    rows_v[e, sl] = rows_v[e, sl] * s
            return 0
        lax.fori_loop(0, C // 16, _scale, 0)

        pltpu.sync_copy(rows_v, acc_sh.at[dst_v.at[k]], add=True)
        return 0
    lax.fori_loop(0, NCHUNK_H, _chunk, 0)

    plsc.subcore_barrier()

    off = 0
    while off < RPT:
        n = min(C, RPT - off)
        pltpu.sync_copy(acc_sh.at[pl.ds(zbase + off, n)],
                        rows_v.at[pl.ds(0, n)])
        pltpu.sync_copy(rows_v.at[pl.ds(0, n)],
                        out_hbm.at[cid].at[pl.ds(zbase + off, n)])
        off += n


_spmm = functools.partial(
    pl.kernel,
    out_type=jax.ShapeDtypeStruct((NC, ACC_ROWS, D), jnp.float32),
    mesh=plsc.VectorSubcoreMesh(core_axis_name="c", subcore_axis_name="s"),
    scratch_types=[
        pltpu.VMEM((NCHUNK_H, C), jnp.int32),
        pltpu.VMEM((NCHUNK_H, C), jnp.int32),
        pltpu.VMEM((NCHUNK_H, C), jnp.float32),
        pltpu.VMEM((C, D), jnp.float32),
        pltpu.VMEM_SHARED((ACC_ROWS, D), jnp.float32),
        pltpu.SemaphoreType.DMA,
    ],
)(_spmm_body)


def _combine_body(p_ref, o_ref):
    o_ref[...] = jnp.maximum(p_ref[0] + p_ref[1], 0.0)


def _combine(partial):
    m_blk = 2000
    return pl.pallas_call(
        _combine_body,
        grid=(N_NODES // m_blk,),
        in_specs=[pl.BlockSpec((NC, m_blk, D), lambda i: (0, i, 0))],
        out_specs=pl.BlockSpec((m_blk, D), lambda i: (i, 0)),
        out_shape=jax.ShapeDtypeStruct((N_NODES, D), jnp.float32),
    )(partial)


def kernel(input, edge_index, edge_weight, weight):
    support = _matmul(input.astype(jnp.float32), weight.astype(jnp.float32))

    pad = E_PAD - N_EDGES
    src = jnp.pad(edge_index[0].astype(jnp.int32), (0, pad)).reshape(NW, NCHUNK, C)
    dst = jnp.pad(edge_index[1].astype(jnp.int32), (0, pad)).reshape(NW, NCHUNK, C)
    ew = jnp.pad(edge_weight.astype(jnp.float32), (0, pad)).reshape(NW, NCHUNK, C)

    partial = _spmm(support, src, dst, ew)
    return _combine(partial)
